# baseline (speedup 1.0000x reference)
import functools

import jax
import jax.numpy as jnp
from jax import lax
from jax.experimental import pallas as pl
from jax.experimental.pallas import tpu as pltpu
from jax.experimental.pallas import tpu_sc as plsc

N = 1024
C = 64
D = 16
NF = 4
CF = C * NF
W = 128
P_PAD = 16
C_OUT = 64
BLK = 128

_NC, _NS = 2, 16
_NW = _NC * _NS
_RPW = (N * D) // _NW
_JCH = _RPW // 128


def _sc_gather_body(conn_ref, tab_ref, g_out, idx_v, idx2_v, g_v, sem):
    wid = lax.axis_index("s") * _NC + lax.axis_index("c")
    pltpu.sync_copy(conn_ref.at[pl.ds(wid * _JCH, _JCH)], idx_v)
    nbase = (wid % 2) * _RPW
    lanes = lax.iota(jnp.int32, 16)
    for i in range(_RPW // 16):
        j, q = divmod(i, 8)
        chunk = idx_v[j, pl.ds(q * 16, 16)]
        nodes = nbase + i * 16 + lanes
        idx2_v[j, pl.ds(q * 16, 16)] = jnp.where(
            chunk == nodes, jnp.int32(N), chunk)
    copies = []
    for j in range(_JCH):
        copies.append(pltpu.async_copy(
            tab_ref.at[idx2_v.at[j]], g_v.at[pl.ds(j * 128, 128)], sem))
    for cp in copies:
        cp.wait()
    pltpu.sync_copy(g_v, g_out.at[pl.ds(wid * _RPW, _RPW)])


def _tab_body(f_ref, p_ref, tab_ref):
    pv = p_ref[...]
    pz = jnp.concatenate(
        [pv, jnp.zeros((N, P_PAD - 3), jnp.float32)], axis=1)
    row = jnp.concatenate(
        [f_ref[...], pz, jnp.zeros((N, W - C - P_PAD), jnp.float32)], axis=1)
    tab_ref[:N, :] = row
    tab_ref[N:, :] = jnp.zeros((8, W), jnp.float32)


def _tc_body(g_ref, ts_ref, wd_ref, bd_ref, wf_ref, bf_ref, out_ref):
    ts = ts_ref[...]
    ps = ts[:, C:C + P_PAD]
    wd = wd_ref[...]
    bd = bd_ref[...]
    acc = jnp.zeros((BLK, CF), jnp.float32)
    for d in range(D):
        row = g_ref[d]
        dirv = row[:, C:C + P_PAD] - ps
        aff = jnp.maximum(
            jnp.dot(dirv, wd, preferred_element_type=jnp.float32) + bd, 0.0)
        fg = row[:, :C]
        stacked = jnp.concatenate([fg] * NF, axis=1)
        acc = acc + stacked * aff
    fs = ts[:, :C]
    acc = acc + jnp.concatenate([fs] * NF, axis=1) * jnp.maximum(bd, 0.0)
    out = jnp.maximum(
        jnp.dot(acc, wf_ref[...], preferred_element_type=jnp.float32)
        + bf_ref[...], 0.0)
    out_ref[...] = out


def kernel(node_feats, node_connections, node_positions, Wd, bd, Wf, bf):
    f32 = jnp.float32
    feats = node_feats[0].astype(f32)
    pos = node_positions[0].astype(f32)
    conn = node_connections.astype(jnp.int32)

    tab = pl.pallas_call(
        _tab_body,
        out_shape=jax.ShapeDtypeStruct((N + 8, W), f32),
    )(feats, pos)
    conn_dmaj = conn.T.reshape(-1, 128)

    mesh = plsc.VectorSubcoreMesh(core_axis_name="c", subcore_axis_name="s")
    sc_gather = functools.partial(
        pl.kernel, mesh=mesh,
        out_type=jax.ShapeDtypeStruct((N * D, W), f32),
        scratch_types=[pltpu.VMEM((_JCH, 128), jnp.int32),
                       pltpu.VMEM((_JCH, 128), jnp.int32),
                       pltpu.VMEM((_RPW, W), f32),
                       pltpu.SemaphoreType.DMA],
    )(_sc_gather_body)
    g = sc_gather(conn_dmaj, tab)

    g3 = g.reshape(D, N, W)
    wd_p = jnp.pad(Wd.T.astype(f32), ((0, P_PAD - Wd.shape[1]), (0, 0)))
    bd2 = bd.astype(f32).reshape(1, CF)
    wf_t = Wf.T.astype(f32)
    bf2 = bf.astype(f32).reshape(1, C_OUT)

    out = pl.pallas_call(
        _tc_body,
        grid=(N // BLK,),
        in_specs=[
            pl.BlockSpec((D, BLK, W), lambda i: (0, i, 0)),
            pl.BlockSpec((BLK, W), lambda i: (i, 0)),
            pl.BlockSpec((P_PAD, CF), lambda i: (0, 0)),
            pl.BlockSpec((1, CF), lambda i: (0, 0)),
            pl.BlockSpec((CF, C_OUT), lambda i: (0, 0)),
            pl.BlockSpec((1, C_OUT), lambda i: (0, 0)),
        ],
        out_specs=pl.BlockSpec((BLK, C_OUT), lambda i: (i, 0)),
        out_shape=jax.ShapeDtypeStruct((N, C_OUT), f32),
        compiler_params=pltpu.CompilerParams(
            dimension_semantics=("arbitrary",)),
    )(g3, tab, wd_p, bd2, wf_t, bf2)
    return out[None]


# speedup vs baseline: 1.0627x; 1.0627x over previous
import functools

import jax
import jax.numpy as jnp
from jax import lax
from jax.experimental import pallas as pl
from jax.experimental.pallas import tpu as pltpu
from jax.experimental.pallas import tpu_sc as plsc

N = 1024
C = 64
D = 16
NF = 4
CF = C * NF
W = 128
P_PAD = 16
C_OUT = 64
BLK = 128

_NC, _NS = 2, 16
_NW = _NC * _NS
_RPW = (N * D) // _NW
_JCH = _RPW // 128


def _sc_gather_body(conn_ref, tab_ref, g_out, idx_v, idx2_v, g_v, sem):
    wid = lax.axis_index("s") * _NC + lax.axis_index("c")
    pltpu.sync_copy(conn_ref.at[pl.ds(wid * _JCH, _JCH)], idx_v)
    nbase = (wid % 2) * _RPW
    lanes = lax.iota(jnp.int32, 16)
    for i in range(_RPW // 16):
        j, q = divmod(i, 8)
        chunk = idx_v[j, pl.ds(q * 16, 16)]
        nodes = nbase + i * 16 + lanes
        idx2_v[j, pl.ds(q * 16, 16)] = jnp.where(
            chunk == nodes, jnp.int32(N), chunk)
    copies = []
    for j in range(_JCH):
        copies.append(pltpu.async_copy(
            tab_ref.at[idx2_v.at[j]], g_v.at[pl.ds(j * 128, 128)], sem))
    for cp in copies:
        cp.wait()
    pltpu.sync_copy(g_v, g_out.at[pl.ds(wid * _RPW, _RPW)])


def _tc_body(g_ref, ps_ref, fs_ref, wd_ref, bd_ref, wf_ref, bf_ref, out_ref):
    ps = ps_ref[...]
    wd = wd_ref[...]
    bd = bd_ref[...]
    acc = jnp.zeros((BLK, CF), jnp.float32)
    for d in range(D):
        row = g_ref[d]
        dirv = row[:, C:C + P_PAD] - ps
        aff = jnp.maximum(
            jnp.dot(dirv, wd, preferred_element_type=jnp.float32) + bd, 0.0)
        fg = row[:, :C]
        stacked = jnp.concatenate([fg] * NF, axis=1)
        acc = acc + stacked * aff
    fs = fs_ref[...]
    acc = acc + jnp.concatenate([fs] * NF, axis=1) * jnp.maximum(bd, 0.0)
    out = jnp.maximum(
        jnp.dot(acc, wf_ref[...], preferred_element_type=jnp.float32)
        + bf_ref[...], 0.0)
    out_ref[...] = out


def kernel(node_feats, node_connections, node_positions, Wd, bd, Wf, bf):
    f32 = jnp.float32
    feats = node_feats[0].astype(f32)
    pos = node_positions[0].astype(f32)
    conn = node_connections.astype(jnp.int32)

    pos_tab = jnp.pad(pos, ((0, 0), (0, P_PAD - pos.shape[1])))
    tab = jnp.concatenate(
        [feats, pos_tab, jnp.zeros((N, W - C - P_PAD), f32)], axis=1)
    tab = jnp.concatenate([tab, jnp.zeros((8, W), f32)], axis=0)
    conn_dmaj = conn.T.reshape(-1, 128)

    mesh = plsc.VectorSubcoreMesh(core_axis_name="c", subcore_axis_name="s")
    sc_gather = functools.partial(
        pl.kernel, mesh=mesh,
        out_type=jax.ShapeDtypeStruct((N * D, W), f32),
        scratch_types=[pltpu.VMEM((_JCH, 128), jnp.int32),
                       pltpu.VMEM((_JCH, 128), jnp.int32),
                       pltpu.VMEM((_RPW, W), f32),
                       pltpu.SemaphoreType.DMA],
    )(_sc_gather_body)
    g = sc_gather(conn_dmaj, tab)

    g3 = g.reshape(D, N, W)
    wd_p = jnp.pad(Wd.T.astype(f32), ((0, P_PAD - Wd.shape[1]), (0, 0)))
    bd2 = bd.astype(f32).reshape(1, CF)
    wf_t = Wf.T.astype(f32)
    bf2 = bf.astype(f32).reshape(1, C_OUT)

    out = pl.pallas_call(
        _tc_body,
        grid=(N // BLK,),
        in_specs=[
            pl.BlockSpec((D, BLK, W), lambda i: (0, i, 0)),
            pl.BlockSpec((BLK, P_PAD), lambda i: (i, 0)),
            pl.BlockSpec((BLK, C), lambda i: (i, 0)),
            pl.BlockSpec((P_PAD, CF), lambda i: (0, 0)),
            pl.BlockSpec((1, CF), lambda i: (0, 0)),
            pl.BlockSpec((CF, C_OUT), lambda i: (0, 0)),
            pl.BlockSpec((1, C_OUT), lambda i: (0, 0)),
        ],
        out_specs=pl.BlockSpec((BLK, C_OUT), lambda i: (i, 0)),
        out_shape=jax.ShapeDtypeStruct((N, C_OUT), f32),
        compiler_params=pltpu.CompilerParams(
            dimension_semantics=("arbitrary",)),
    )(g3, pos_tab, feats, wd_p, bd2, wf_t, bf2)
    return out[None]
